# Initial kernel scaffold; baseline (speedup 1.0000x reference)
import jax
import jax.numpy as jnp
from jax.experimental import pallas as pl


def kernel(x, edge_index, edge_attr, W_e, b_e, eps, W1, b1, bn_gamma, bn_beta, W2, b2, ln_gamma, ln_beta):
    raise NotImplementedError("write your pallas kernel here")


# speedup vs baseline: 1.2884x; 1.2884x over previous
import functools

import jax
import jax.numpy as jnp
from jax import lax
from jax.experimental import pallas as pl
from jax.experimental.pallas import tpu as pltpu, tpu_sc as plsc

N = 10000
D = 128
DE = 16
E = 320000

NC = 2
NS = 16
CHUNK = 128
CPT = 160
E_PAD = CPT * CHUNK * NS
N_HALF = 5120
JUNK = N_HALF
ACC_ROWS = N_HALF + 8
ROWS_PER_TILE = N_HALF // NS


def _eproj_body(ea_ref, we_ref, be_ref, o_ref):
    o_ref[...] = (
        jnp.dot(ea_ref[...], we_ref[...], preferred_element_type=jnp.float32)
        + be_ref[...]
    )


def _edge_proj(ea_pad, W_e, b_e):
    BE = 1024
    grid = (E_PAD // BE,)
    return pl.pallas_call(
        _eproj_body,
        grid=grid,
        in_specs=[
            pl.BlockSpec((BE, DE), lambda i: (i, 0)),
            pl.BlockSpec((DE, D), lambda i: (0, 0)),
            pl.BlockSpec((1, D), lambda i: (0, 0)),
        ],
        out_specs=pl.BlockSpec((BE, D), lambda i: (i, 0)),
        out_shape=jax.ShapeDtypeStruct((E_PAD, D), jnp.float32),
    )(ea_pad, W_e, b_e.reshape(1, D))


def _sc_body(x_hbm, eproj_hbm, srcs_hbm, dsts_hbm, out_hbm,
             src_v, dst_v, e_buf, g_buf, agg_sh, sem_e, sem_g):
    c = lax.axis_index("c")
    s = lax.axis_index("s")

    def _zrow(r, carry):
        for k in range(D // 16):
            g_buf[r, pl.ds(k * 16, 16)] = jnp.zeros((16,), jnp.float32)
        return carry

    lax.fori_loop(0, CHUNK, _zrow, 0)
    base = s * ROWS_PER_TILE
    full, rem = divmod(ROWS_PER_TILE, CHUNK)
    for k in range(full):
        pltpu.sync_copy(g_buf, agg_sh.at[pl.ds(base + k * CHUNK, CHUNK)])
    if rem:
        pltpu.sync_copy(g_buf.at[pl.ds(0, rem)],
                        agg_sh.at[pl.ds(base + full * CHUNK, rem)])
    @pl.when(s == 0)
    def _():
        pltpu.sync_copy(g_buf.at[pl.ds(0, ACC_ROWS - N_HALF)],
                        agg_sh.at[pl.ds(N_HALF, ACC_ROWS - N_HALF)])

    plsc.subcore_barrier()

    pltpu.sync_copy(srcs_hbm.at[pl.ds(s * CPT, CPT)], src_v)
    pltpu.sync_copy(dsts_hbm.at[pl.ds(s * CPT, CPT)], dst_v)

    def _remap(lo):
        def _rrow(r, carry):
            for k in range(CHUNK // 16):
                sl = pl.ds(k * 16, 16)
                t = dst_v[r, sl] - lo
                ok = (t >= 0) & (t < N_HALF)
                dst_v[r, sl] = jnp.where(ok, t, JUNK)
            return carry

        lax.fori_loop(0, CPT, _rrow, 0)

    @pl.when(c == 0)
    def _():
        _remap(0)

    @pl.when(c == 1)
    def _():
        _remap(N_HALF)

    def _chunk(j, carry):
        gcid = s * CPT + j
        cp_e = pltpu.async_copy(eproj_hbm.at[pl.ds(gcid * CHUNK, CHUNK)],
                                e_buf, sem_e)
        cp_g = pltpu.async_copy(x_hbm.at[src_v.at[j]], g_buf, sem_g)
        cp_e.wait()
        cp_g.wait()

        def _row(r, rc):
            for k in range(D // 16):
                sl = pl.ds(k * 16, 16)
                g_buf[r, sl] = jnp.maximum(g_buf[r, sl] + e_buf[r, sl], 0.0)
            return rc

        lax.fori_loop(0, CHUNK, _row, 0)
        pltpu.sync_copy(g_buf, agg_sh.at[dst_v.at[j]], add=True)
        return carry

    lax.fori_loop(0, CPT, _chunk, 0)
    plsc.subcore_barrier()

    pltpu.sync_copy(agg_sh.at[pl.ds(base, ROWS_PER_TILE)],
                    out_hbm.at[c].at[pl.ds(base, ROWS_PER_TILE)])


def _sc_aggregate(x, eproj, srcs2d, dsts2d):
    mesh = plsc.VectorSubcoreMesh(core_axis_name="c", subcore_axis_name="s",
                                  num_cores=NC, num_subcores=NS)
    fn = pl.kernel(
        _sc_body,
        out_type=jax.ShapeDtypeStruct((NC, N_HALF, D), jnp.float32),
        mesh=mesh,
        scratch_types=[
            pltpu.VMEM((CPT, CHUNK), jnp.int32),
            pltpu.VMEM((CPT, CHUNK), jnp.int32),
            pltpu.VMEM((CHUNK, D), jnp.float32),
            pltpu.VMEM((CHUNK, D), jnp.float32),
            pltpu.VMEM_SHARED((ACC_ROWS, D), jnp.float32),
            pltpu.SemaphoreType.DMA,
            pltpu.SemaphoreType.DMA,
        ],
    )
    return fn(x, eproj, srcs2d, dsts2d)


def _mlp_body(eps_ref, x_ref, a_ref, w1_ref, b1_ref, w2_ref, b2_ref,
              lng_ref, lnb_ref, o_ref):
    h = x_ref[...] * eps_ref[0] + a_ref[...]
    h1 = jnp.dot(h, w1_ref[...], preferred_element_type=jnp.float32) + b1_ref[...]
    h1 = jnp.maximum(h1, 0.0)
    h2 = jnp.dot(h1, w2_ref[...], preferred_element_type=jnp.float32) + b2_ref[...]
    mu = jnp.mean(h2, axis=1, keepdims=True)
    dc = h2 - mu
    var = jnp.mean(dc * dc, axis=1, keepdims=True)
    hn = dc * lax.rsqrt(var + 1e-5) * lng_ref[...] + lnb_ref[...]
    o_ref[...] = jnp.maximum(hn + x_ref[...], 0.0)


def _node_mlp(epsv, x, agg, W1f, b1f, W2, b2, ln_gamma, ln_beta):
    BN = 1000
    grid = (N // BN,)
    vec = lambda a: a.reshape(1, D)
    return pl.pallas_call(
        _mlp_body,
        grid=grid,
        in_specs=[
            pl.BlockSpec(memory_space=pltpu.SMEM),
            pl.BlockSpec((BN, D), lambda i: (i, 0)),
            pl.BlockSpec((BN, D), lambda i: (i, 0)),
            pl.BlockSpec((D, D), lambda i: (0, 0)),
            pl.BlockSpec((1, D), lambda i: (0, 0)),
            pl.BlockSpec((D, D), lambda i: (0, 0)),
            pl.BlockSpec((1, D), lambda i: (0, 0)),
            pl.BlockSpec((1, D), lambda i: (0, 0)),
            pl.BlockSpec((1, D), lambda i: (0, 0)),
        ],
        out_specs=pl.BlockSpec((BN, D), lambda i: (i, 0)),
        out_shape=jax.ShapeDtypeStruct((N, D), jnp.float32),
    )(epsv, x, agg, W1f, vec(b1f), W2, vec(b2), vec(ln_gamma), vec(ln_beta))


def kernel(x, edge_index, edge_attr, W_e, b_e, eps, W1, b1, bn_gamma, bn_beta,
           W2, b2, ln_gamma, ln_beta):
    src = edge_index[0].astype(jnp.int32)
    dst = edge_index[1].astype(jnp.int32)
    pad = E_PAD - E
    src_p = jnp.concatenate([src, jnp.zeros((pad,), jnp.int32)])
    dst_p = jnp.concatenate([dst, jnp.full((pad,), N, jnp.int32)])
    srcs2d = src_p.reshape(E_PAD // CHUNK, CHUNK)
    dsts2d = dst_p.reshape(E_PAD // CHUNK, CHUNK)
    ea_pad = jnp.concatenate(
        [edge_attr, jnp.zeros((pad, DE), edge_attr.dtype)], axis=0)

    eproj = _edge_proj(ea_pad, W_e, b_e)
    partials = _sc_aggregate(x, eproj, srcs2d, dsts2d)
    agg = partials.reshape(NC * N_HALF, D)

    scale = bn_gamma / jnp.sqrt(1.0 + 1e-5)
    W1f = W1 * scale[None, :]
    b1f = b1 * scale + bn_beta
    epsv = (1.0 + eps).reshape(1).astype(jnp.float32)

    return _node_mlp(epsv, x, agg, W1f, b1f, W2, b2, ln_gamma, ln_beta)


# speedup vs baseline: 1.4113x; 1.0954x over previous
import functools

import jax
import jax.numpy as jnp
from jax import lax
from jax.experimental import pallas as pl
from jax.experimental.pallas import tpu as pltpu, tpu_sc as plsc

N = 10000
D = 128
DE = 16
E = 320000

NC = 2
NS = 16
CHUNK = 128
CPT = 160
E_PAD = CPT * CHUNK * NS
N_HALF = 5120
JUNK = N_HALF
ACC_ROWS = N_HALF + 8
ROWS_PER_TILE = N_HALF // NS


def _eproj_body(ea_ref, we_ref, be_ref, o_ref):
    o_ref[...] = (
        jnp.dot(ea_ref[...], we_ref[...], preferred_element_type=jnp.float32)
        + be_ref[...]
    )


def _edge_proj(ea_pad, W_e, b_e):
    BE = 1024
    grid = (E_PAD // BE,)
    return pl.pallas_call(
        _eproj_body,
        grid=grid,
        in_specs=[
            pl.BlockSpec((BE, DE), lambda i: (i, 0)),
            pl.BlockSpec((DE, D), lambda i: (0, 0)),
            pl.BlockSpec((1, D), lambda i: (0, 0)),
        ],
        out_specs=pl.BlockSpec((BE, D), lambda i: (i, 0)),
        out_shape=jax.ShapeDtypeStruct((E_PAD, D), jnp.float32),
    )(ea_pad, W_e, b_e.reshape(1, D))


GRP = 8
NGRP = CPT // GRP


def _sc_body(x_hbm, eproj_hbm, srcs_hbm, dsts_hbm, out_hbm,
             src8, dst8, e_bufs, g_bufs, agg_sh,
             sems_src, sems_dst, sems_e, sems_g, sems_s):
    c = lax.axis_index("c")
    s = lax.axis_index("s")
    lo = c * N_HALF
    g_buf = g_bufs[0]

    def _zrow(r, carry):
        for k in range(D // 16):
            g_buf[r, pl.ds(k * 16, 16)] = jnp.zeros((16,), jnp.float32)
        return carry

    lax.fori_loop(0, CHUNK, _zrow, 0)
    base = s * ROWS_PER_TILE
    full, rem = divmod(ROWS_PER_TILE, CHUNK)
    for k in range(full):
        pltpu.sync_copy(g_buf, agg_sh.at[pl.ds(base + k * CHUNK, CHUNK)])
    if rem:
        pltpu.sync_copy(g_buf.at[pl.ds(0, rem)],
                        agg_sh.at[pl.ds(base + full * CHUNK, rem)])
    @pl.when(s == 0)
    def _():
        pltpu.sync_copy(g_buf.at[pl.ds(0, ACC_ROWS - N_HALF)],
                        agg_sh.at[pl.ds(N_HALF, ACC_ROWS - N_HALF)])

    plsc.subcore_barrier()

    def _issue_idx(grp, i):
        off = s * CPT + grp * GRP
        pltpu.async_copy(srcs_hbm.at[pl.ds(off, GRP)], src8[i], sems_src[i])
        pltpu.async_copy(dsts_hbm.at[pl.ds(off, GRP)], dst8[i], sems_dst[i])

    def _wait_src(grp, i):
        off = s * CPT + grp * GRP
        pltpu.make_async_copy(srcs_hbm.at[pl.ds(off, GRP)], src8[i],
                              sems_src[i]).wait()

    def _wait_dst(grp, i):
        off = s * CPT + grp * GRP
        pltpu.make_async_copy(dsts_hbm.at[pl.ds(off, GRP)], dst8[i],
                              sems_dst[i]).wait()

    def _remap(i):
        for r in range(GRP):
            for k in range(CHUNK // 16):
                sl = pl.ds(k * 16, 16)
                t = dst8[i][r, sl] - lo
                ok = (t >= 0) & (t < N_HALF)
                dst8[i][r, sl] = jnp.where(ok, t, JUNK)

    def _issue_eg(jj, b, src_row):
        gcid = s * CPT + jj
        pltpu.async_copy(eproj_hbm.at[pl.ds(gcid * CHUNK, CHUNK)],
                         e_bufs[b], sems_e[b])
        pltpu.async_copy(x_hbm.at[src_row], g_bufs[b], sems_g[b])

    def _wait_eg(jj, b, src_row):
        gcid = s * CPT + jj
        pltpu.make_async_copy(eproj_hbm.at[pl.ds(gcid * CHUNK, CHUNK)],
                              e_bufs[b], sems_e[b]).wait()
        pltpu.make_async_copy(x_hbm.at[src_row], g_bufs[b], sems_g[b]).wait()

    def _wait_scat(b, dst_row):
        pltpu.make_async_copy(g_bufs[b], agg_sh.at[dst_row], sems_s[b]).wait()

    _issue_idx(0, 0)
    _wait_src(0, 0)
    _wait_dst(0, 0)
    _remap(0)
    _issue_eg(0, 0, src8[0].at[0])

    def _super(p, carry):
        for i in (0, 1):
            grp = 2 * p + i
            if i == 0:
                @pl.when(p > 0)
                def _():
                    _wait_dst(grp, 0)
                    _remap(0)
            else:
                _wait_dst(grp, 1)
                _remap(1)

            for q in range(GRP):
                jj = grp * GRP + q
                b = q % 2
                if q == 2:
                    if i == 0:
                        _issue_idx(grp + 1, 1)
                    else:
                        @pl.when(p < NGRP // 2 - 1)
                        def _():
                            _issue_idx(grp + 1, 0)
                _wait_eg(jj, b, src8[i].at[q])
                if q == 0:
                    if i == 0:
                        @pl.when(p > 0)
                        def _():
                            _wait_scat(1 - b, dst8[1].at[GRP - 1])
                    else:
                        _wait_scat(1 - b, dst8[0].at[GRP - 1])
                else:
                    _wait_scat(1 - b, dst8[i].at[q - 1])
                if q < GRP - 1:
                    _issue_eg(jj + 1, 1 - b, src8[i].at[q + 1])
                else:
                    if i == 0:
                        _wait_src(grp + 1, 1)
                        _issue_eg(jj + 1, 1 - b, src8[1].at[0])
                    else:
                        @pl.when(p < NGRP // 2 - 1)
                        def _():
                            _wait_src(grp + 1, 0)
                            _issue_eg(jj + 1, 1 - b, src8[0].at[0])

                e_buf = e_bufs[b]
                gb = g_bufs[b]

                @plsc.parallel_loop(0, CHUNK, 1, unroll=2)
                def _row(r):
                    for k in range(D // 16):
                        sl = pl.ds(k * 16, 16)
                        gb[r, sl] = jnp.maximum(gb[r, sl] + e_buf[r, sl], 0.0)

                pltpu.async_copy(gb, agg_sh.at[dst8[i].at[q]], sems_s[b],
                                 add=True)
        return carry

    lax.fori_loop(0, NGRP // 2, _super, 0)
    _wait_scat(1, dst8[1].at[GRP - 1])
    plsc.subcore_barrier()

    pltpu.sync_copy(agg_sh.at[pl.ds(base, ROWS_PER_TILE)],
                    out_hbm.at[c].at[pl.ds(base, ROWS_PER_TILE)])


def _sc_aggregate(x, eproj, srcs2d, dsts2d):
    mesh = plsc.VectorSubcoreMesh(core_axis_name="c", subcore_axis_name="s",
                                  num_cores=NC, num_subcores=NS)
    fn = pl.kernel(
        _sc_body,
        out_type=jax.ShapeDtypeStruct((NC, N_HALF, D), jnp.float32),
        mesh=mesh,
        scratch_types=[
            (pltpu.VMEM((GRP, CHUNK), jnp.int32),
             pltpu.VMEM((GRP, CHUNK), jnp.int32)),
            (pltpu.VMEM((GRP, CHUNK), jnp.int32),
             pltpu.VMEM((GRP, CHUNK), jnp.int32)),
            (pltpu.VMEM((CHUNK, D), jnp.float32),
             pltpu.VMEM((CHUNK, D), jnp.float32)),
            (pltpu.VMEM((CHUNK, D), jnp.float32),
             pltpu.VMEM((CHUNK, D), jnp.float32)),
            pltpu.VMEM_SHARED((ACC_ROWS, D), jnp.float32),
            (pltpu.SemaphoreType.DMA, pltpu.SemaphoreType.DMA),
            (pltpu.SemaphoreType.DMA, pltpu.SemaphoreType.DMA),
            (pltpu.SemaphoreType.DMA, pltpu.SemaphoreType.DMA),
            (pltpu.SemaphoreType.DMA, pltpu.SemaphoreType.DMA),
            (pltpu.SemaphoreType.DMA, pltpu.SemaphoreType.DMA),
        ],
    )
    return fn(x, eproj, srcs2d, dsts2d)


def _mlp_body(eps_ref, x_ref, a_ref, w1_ref, b1_ref, w2_ref, b2_ref,
              lng_ref, lnb_ref, o_ref):
    h = x_ref[...] * eps_ref[0] + a_ref[...]
    h1 = jnp.dot(h, w1_ref[...], preferred_element_type=jnp.float32) + b1_ref[...]
    h1 = jnp.maximum(h1, 0.0)
    h2 = jnp.dot(h1, w2_ref[...], preferred_element_type=jnp.float32) + b2_ref[...]
    mu = jnp.mean(h2, axis=1, keepdims=True)
    dc = h2 - mu
    var = jnp.mean(dc * dc, axis=1, keepdims=True)
    hn = dc * lax.rsqrt(var + 1e-5) * lng_ref[...] + lnb_ref[...]
    o_ref[...] = jnp.maximum(hn + x_ref[...], 0.0)


def _node_mlp(epsv, x, agg, W1f, b1f, W2, b2, ln_gamma, ln_beta):
    BN = 1000
    grid = (N // BN,)
    vec = lambda a: a.reshape(1, D)
    return pl.pallas_call(
        _mlp_body,
        grid=grid,
        in_specs=[
            pl.BlockSpec(memory_space=pltpu.SMEM),
            pl.BlockSpec((BN, D), lambda i: (i, 0)),
            pl.BlockSpec((BN, D), lambda i: (i, 0)),
            pl.BlockSpec((D, D), lambda i: (0, 0)),
            pl.BlockSpec((1, D), lambda i: (0, 0)),
            pl.BlockSpec((D, D), lambda i: (0, 0)),
            pl.BlockSpec((1, D), lambda i: (0, 0)),
            pl.BlockSpec((1, D), lambda i: (0, 0)),
            pl.BlockSpec((1, D), lambda i: (0, 0)),
        ],
        out_specs=pl.BlockSpec((BN, D), lambda i: (i, 0)),
        out_shape=jax.ShapeDtypeStruct((N, D), jnp.float32),
    )(epsv, x, agg, W1f, vec(b1f), W2, vec(b2), vec(ln_gamma), vec(ln_beta))


def kernel(x, edge_index, edge_attr, W_e, b_e, eps, W1, b1, bn_gamma, bn_beta,
           W2, b2, ln_gamma, ln_beta):
    src = edge_index[0].astype(jnp.int32)
    dst = edge_index[1].astype(jnp.int32)
    pad = E_PAD - E
    src_p = jnp.concatenate([src, jnp.zeros((pad,), jnp.int32)])
    dst_p = jnp.concatenate([dst, jnp.full((pad,), N, jnp.int32)])
    srcs2d = src_p.reshape(E_PAD // CHUNK, CHUNK)
    dsts2d = dst_p.reshape(E_PAD // CHUNK, CHUNK)
    ea_pad = jnp.concatenate(
        [edge_attr, jnp.zeros((pad, DE), edge_attr.dtype)], axis=0)

    eproj = _edge_proj(ea_pad, W_e, b_e)
    partials = _sc_aggregate(x, eproj, srcs2d, dsts2d)
    agg = partials.reshape(NC * N_HALF, D)

    scale = bn_gamma / jnp.sqrt(1.0 + 1e-5)
    W1f = W1 * scale[None, :]
    b1f = b1 * scale + bn_beta
    epsv = (1.0 + eps).reshape(1).astype(jnp.float32)

    return _node_mlp(epsv, x, agg, W1f, b1f, W2, b2, ln_gamma, ln_beta)


# speedup vs baseline: 1.8146x; 1.2858x over previous
import functools

import jax
import jax.numpy as jnp
from jax import lax
from jax.experimental import pallas as pl
from jax.experimental.pallas import tpu as pltpu, tpu_sc as plsc

N = 10000
D = 128
DE = 16
E = 320000

NC = 2
NS = 16
NW = NC * NS
CHUNK = 128
CPT = 80
E_PAD = CPT * CHUNK * NW
GRP = 8
NGRP = CPT // GRP
ACC_ROWS = N
ROWS_MAIN = 624
ROWS_LAST = N - 15 * ROWS_MAIN


def _eproj_body(ea_ref, we_ref, be_ref, o_ref):
    i = pl.program_id(0)
    e = (
        jnp.dot(ea_ref[...], we_ref[...], preferred_element_type=jnp.float32)
        + be_ref[...]
    )
    rows = lax.broadcasted_iota(jnp.int32, (e.shape[0], 1), 0) + i * e.shape[0]
    o_ref[...] = jnp.where(rows < E, e, -1e30)


def _edge_proj(ea_pad, W_e, b_e):
    BE = 1024
    grid = (E_PAD // BE,)
    return pl.pallas_call(
        _eproj_body,
        grid=grid,
        in_specs=[
            pl.BlockSpec((BE, DE), lambda i: (i, 0)),
            pl.BlockSpec((DE, D), lambda i: (0, 0)),
            pl.BlockSpec((1, D), lambda i: (0, 0)),
        ],
        out_specs=pl.BlockSpec((BE, D), lambda i: (i, 0)),
        out_shape=jax.ShapeDtypeStruct((E_PAD, D), jnp.float32),
    )(ea_pad, W_e, b_e.reshape(1, D))


def _sc_body(x_hbm, eproj_hbm, srcs_hbm, dsts_hbm, out_hbm,
             src8, dst8, e_buf, g_buf, agg_sh,
             sem_e, sem_g, sem_s):
    c = lax.axis_index("c")
    s = lax.axis_index("s")
    wid = s * NC + c

    z_buf = g_buf

    def _zrow(r, carry):
        for k in range(D // 16):
            z_buf[r, pl.ds(k * 16, 16)] = jnp.zeros((16,), jnp.float32)
        return carry

    lax.fori_loop(0, CHUNK, _zrow, 0)
    zbase = s * ROWS_MAIN

    @pl.when(s < NS - 1)
    def _():
        for k in range(4):
            cnt = CHUNK if k < 4 - 1 else ROWS_MAIN - 3 * CHUNK
            pltpu.sync_copy(z_buf.at[pl.ds(0, cnt)],
                            agg_sh.at[pl.ds(zbase + k * CHUNK, cnt)])

    @pl.when(s == NS - 1)
    def _():
        for k in range(5):
            cnt = CHUNK if k < 5 - 1 else ROWS_LAST - 4 * CHUNK
            pltpu.sync_copy(z_buf.at[pl.ds(0, cnt)],
                            agg_sh.at[pl.ds(zbase + k * CHUNK, cnt)])

    plsc.subcore_barrier()

    def _issue_g(src_row):
        pltpu.async_copy(x_hbm.at[src_row], g_buf, sem_g)

    def _wait_g(src_row):
        pltpu.make_async_copy(x_hbm.at[src_row], g_buf, sem_g).wait()

    def _eproj_rows(jj):
        return eproj_hbm.at[pl.ds((wid * CPT + jj) * CHUNK, CHUNK)]

    def _issue_e(jj):
        pltpu.async_copy(_eproj_rows(jj), e_buf, sem_e)

    def _wait_e(jj):
        pltpu.make_async_copy(_eproj_rows(jj), e_buf, sem_e).wait()

    def _wait_scat(dst_row):
        pltpu.make_async_copy(e_buf, agg_sh.at[dst_row], sem_s).wait()

    _issue_e(0)

    def _group(p, carry):
        @pl.when(p > 0)
        def _():
            _wait_scat(dst8.at[GRP - 1])
            _issue_e(p * GRP)

        off = wid * CPT + p * GRP
        pltpu.sync_copy(srcs_hbm.at[pl.ds(off, GRP)], src8)
        pltpu.sync_copy(dsts_hbm.at[pl.ds(off, GRP)], dst8)
        _issue_g(src8.at[0])

        for q in range(GRP):
            _wait_g(src8.at[q])
            _wait_e(p * GRP + q)

            @plsc.parallel_loop(0, CHUNK, 1, unroll=2)
            def _row(r):
                for k in range(D // 16):
                    sl = pl.ds(k * 16, 16)
                    e_buf[r, sl] = jnp.maximum(g_buf[r, sl] + e_buf[r, sl],
                                               0.0)

            pltpu.async_copy(e_buf, agg_sh.at[dst8.at[q]], sem_s, add=True)
            if q < GRP - 1:
                _issue_g(src8.at[q + 1])
                _wait_scat(dst8.at[q])
                _issue_e(p * GRP + q + 1)

        return carry

    lax.fori_loop(0, NGRP, _group, 0)
    _wait_scat(dst8.at[GRP - 1])
    plsc.subcore_barrier()

    @pl.when(s < NS - 1)
    def _():
        pltpu.sync_copy(agg_sh.at[pl.ds(zbase, ROWS_MAIN)],
                        out_hbm.at[c].at[pl.ds(zbase, ROWS_MAIN)])

    @pl.when(s == NS - 1)
    def _():
        pltpu.sync_copy(agg_sh.at[pl.ds(zbase, ROWS_LAST)],
                        out_hbm.at[c].at[pl.ds(zbase, ROWS_LAST)])


def _sc_aggregate(x, eproj, srcs2d, dsts2d):
    mesh = plsc.VectorSubcoreMesh(core_axis_name="c", subcore_axis_name="s",
                                  num_cores=NC, num_subcores=NS)
    fn = pl.kernel(
        _sc_body,
        out_type=jax.ShapeDtypeStruct((NC, N, D), jnp.float32),
        mesh=mesh,
        scratch_types=[
            pltpu.VMEM((GRP, CHUNK), jnp.int32),
            pltpu.VMEM((GRP, CHUNK), jnp.int32),
            pltpu.VMEM((CHUNK, D), jnp.float32),
            pltpu.VMEM((CHUNK, D), jnp.float32),
            pltpu.VMEM_SHARED((ACC_ROWS, D), jnp.float32),
            pltpu.SemaphoreType.DMA,
            pltpu.SemaphoreType.DMA,
            pltpu.SemaphoreType.DMA,
        ],
    )
    return fn(x, eproj, srcs2d, dsts2d)


def _mlp_body(eps_ref, x_ref, a0_ref, a1_ref, w1_ref, b1_ref, w2_ref, b2_ref,
              lng_ref, lnb_ref, o_ref):
    h = x_ref[...] * eps_ref[0] + a0_ref[...] + a1_ref[...]
    h1 = jnp.dot(h, w1_ref[...], preferred_element_type=jnp.float32) + b1_ref[...]
    h1 = jnp.maximum(h1, 0.0)
    h2 = jnp.dot(h1, w2_ref[...], preferred_element_type=jnp.float32) + b2_ref[...]
    mu = jnp.mean(h2, axis=1, keepdims=True)
    dc = h2 - mu
    var = jnp.mean(dc * dc, axis=1, keepdims=True)
    hn = dc * lax.rsqrt(var + 1e-5) * lng_ref[...] + lnb_ref[...]
    o_ref[...] = jnp.maximum(hn + x_ref[...], 0.0)


def _node_mlp(epsv, x, a0, a1, W1f, b1f, W2, b2, ln_gamma, ln_beta):
    BN = 1000
    grid = (N // BN,)
    vec = lambda a: a.reshape(1, D)
    return pl.pallas_call(
        _mlp_body,
        grid=grid,
        in_specs=[
            pl.BlockSpec(memory_space=pltpu.SMEM),
            pl.BlockSpec((BN, D), lambda i: (i, 0)),
            pl.BlockSpec((BN, D), lambda i: (i, 0)),
            pl.BlockSpec((BN, D), lambda i: (i, 0)),
            pl.BlockSpec((D, D), lambda i: (0, 0)),
            pl.BlockSpec((1, D), lambda i: (0, 0)),
            pl.BlockSpec((D, D), lambda i: (0, 0)),
            pl.BlockSpec((1, D), lambda i: (0, 0)),
            pl.BlockSpec((1, D), lambda i: (0, 0)),
            pl.BlockSpec((1, D), lambda i: (0, 0)),
        ],
        out_specs=pl.BlockSpec((BN, D), lambda i: (i, 0)),
        out_shape=jax.ShapeDtypeStruct((N, D), jnp.float32),
    )(epsv, x, a0, a1, W1f, vec(b1f), W2, vec(b2), vec(ln_gamma), vec(ln_beta))


def kernel(x, edge_index, edge_attr, W_e, b_e, eps, W1, b1, bn_gamma, bn_beta,
           W2, b2, ln_gamma, ln_beta):
    src = edge_index[0].astype(jnp.int32)
    dst = edge_index[1].astype(jnp.int32)
    pad = E_PAD - E
    src_p = jnp.concatenate([src, jnp.zeros((pad,), jnp.int32)])
    dst_p = jnp.concatenate([dst, jnp.zeros((pad,), jnp.int32)])
    srcs2d = src_p.reshape(E_PAD // CHUNK, CHUNK)
    dsts2d = dst_p.reshape(E_PAD // CHUNK, CHUNK)
    ea_pad = jnp.concatenate(
        [edge_attr, jnp.zeros((pad, DE), edge_attr.dtype)], axis=0)

    eproj = _edge_proj(ea_pad, W_e, b_e)
    partials = _sc_aggregate(x, eproj, srcs2d, dsts2d)
    a0 = partials[0]
    a1 = partials[1]

    scale = bn_gamma / jnp.sqrt(1.0 + 1e-5)
    W1f = W1 * scale[None, :]
    b1f = b1 * scale + bn_beta
    epsv = (1.0 + eps).reshape(1).astype(jnp.float32)

    return _node_mlp(epsv, x, a0, a1, W1f, b1f, W2, b2, ln_gamma, ln_beta)


# speedup vs baseline: 1.8166x; 1.0011x over previous
import functools

import jax
import jax.numpy as jnp
from jax import lax
from jax.experimental import pallas as pl
from jax.experimental.pallas import tpu as pltpu, tpu_sc as plsc

N = 10000
D = 128
DE = 16
E = 320000

NC = 2
NS = 16
NW = NC * NS
CHUNK = 128
CPT = 80
E_PAD = CPT * CHUNK * NW
GRP = 8
NGRP = CPT // GRP
ACC_ROWS = N
ROWS_MAIN = 624
ROWS_LAST = N - 15 * ROWS_MAIN


def _eproj_body(ea_ref, we_ref, be_ref, o_ref):
    i = pl.program_id(0)
    e = (
        jnp.dot(ea_ref[...], we_ref[...], preferred_element_type=jnp.float32)
        + be_ref[...]
    )
    rows = lax.broadcasted_iota(jnp.int32, (e.shape[0], 1), 0) + i * e.shape[0]
    o_ref[...] = jnp.where(rows < E, e, -1e30)


def _edge_proj(ea_pad, W_e, b_e):
    BE = 1024
    grid = (E_PAD // BE,)
    return pl.pallas_call(
        _eproj_body,
        grid=grid,
        in_specs=[
            pl.BlockSpec((BE, DE), lambda i: (i, 0)),
            pl.BlockSpec((DE, D), lambda i: (0, 0)),
            pl.BlockSpec((1, D), lambda i: (0, 0)),
        ],
        out_specs=pl.BlockSpec((BE, D), lambda i: (i, 0)),
        out_shape=jax.ShapeDtypeStruct((E_PAD, D), jnp.float32),
    )(ea_pad, W_e, b_e.reshape(1, D))


def _sc_body(x_hbm, eproj_hbm, srcs_hbm, dsts_hbm, out_hbm,
             src8, dst8, e_buf, g_buf, agg_sh,
             sem_e, sem_g, sem_s):
    c = lax.axis_index("c")
    s = lax.axis_index("s")
    wid = s * NC + c

    z_buf = g_buf

    def _zrow(r, carry):
        for k in range(D // 16):
            z_buf[r, pl.ds(k * 16, 16)] = jnp.zeros((16,), jnp.float32)
        return carry

    lax.fori_loop(0, CHUNK, _zrow, 0)
    zbase = s * ROWS_MAIN

    @pl.when(s < NS - 1)
    def _():
        for k in range(4):
            cnt = CHUNK if k < 4 - 1 else ROWS_MAIN - 3 * CHUNK
            pltpu.sync_copy(z_buf.at[pl.ds(0, cnt)],
                            agg_sh.at[pl.ds(zbase + k * CHUNK, cnt)])

    @pl.when(s == NS - 1)
    def _():
        for k in range(5):
            cnt = CHUNK if k < 5 - 1 else ROWS_LAST - 4 * CHUNK
            pltpu.sync_copy(z_buf.at[pl.ds(0, cnt)],
                            agg_sh.at[pl.ds(zbase + k * CHUNK, cnt)])

    plsc.subcore_barrier()

    def _issue_g(src_row):
        pltpu.async_copy(x_hbm.at[src_row], g_buf, sem_g)

    def _wait_g(src_row):
        pltpu.make_async_copy(x_hbm.at[src_row], g_buf, sem_g).wait()

    def _eproj_rows(jj):
        return eproj_hbm.at[pl.ds((wid * CPT + jj) * CHUNK, CHUNK)]

    def _issue_e(jj):
        pltpu.async_copy(_eproj_rows(jj), e_buf, sem_e)

    def _wait_e(jj):
        pltpu.make_async_copy(_eproj_rows(jj), e_buf, sem_e).wait()

    def _wait_scat(dst_row):
        pltpu.make_async_copy(e_buf, agg_sh.at[dst_row], sem_s).wait()

    _issue_e(0)

    def _group(p, carry):
        @pl.when(p > 0)
        def _():
            _wait_scat(dst8.at[GRP - 1])
            _issue_e(p * GRP)

        off = wid * CPT + p * GRP
        pltpu.sync_copy(srcs_hbm.at[pl.ds(off, GRP)], src8)
        pltpu.sync_copy(dsts_hbm.at[pl.ds(off, GRP)], dst8)
        _issue_g(src8.at[0])

        for q in range(GRP):
            _wait_g(src8.at[q])
            _wait_e(p * GRP + q)

            @plsc.parallel_loop(0, CHUNK, 1, unroll=4)
            def _row(r):
                for k in range(D // 16):
                    sl = pl.ds(k * 16, 16)
                    e_buf[r, sl] = jnp.maximum(g_buf[r, sl] + e_buf[r, sl],
                                               0.0)

            pltpu.async_copy(e_buf, agg_sh.at[dst8.at[q]], sem_s, add=True)
            if q < GRP - 1:
                _issue_g(src8.at[q + 1])
                _wait_scat(dst8.at[q])
                _issue_e(p * GRP + q + 1)

        return carry

    lax.fori_loop(0, NGRP, _group, 0)
    _wait_scat(dst8.at[GRP - 1])
    plsc.subcore_barrier()

    @pl.when(s < NS - 1)
    def _():
        pltpu.sync_copy(agg_sh.at[pl.ds(zbase, ROWS_MAIN)],
                        out_hbm.at[c].at[pl.ds(zbase, ROWS_MAIN)])

    @pl.when(s == NS - 1)
    def _():
        pltpu.sync_copy(agg_sh.at[pl.ds(zbase, ROWS_LAST)],
                        out_hbm.at[c].at[pl.ds(zbase, ROWS_LAST)])


def _sc_aggregate(x, eproj, srcs2d, dsts2d):
    mesh = plsc.VectorSubcoreMesh(core_axis_name="c", subcore_axis_name="s",
                                  num_cores=NC, num_subcores=NS)
    fn = pl.kernel(
        _sc_body,
        out_type=jax.ShapeDtypeStruct((NC, N, D), jnp.float32),
        mesh=mesh,
        scratch_types=[
            pltpu.VMEM((GRP, CHUNK), jnp.int32),
            pltpu.VMEM((GRP, CHUNK), jnp.int32),
            pltpu.VMEM((CHUNK, D), jnp.float32),
            pltpu.VMEM((CHUNK, D), jnp.float32),
            pltpu.VMEM_SHARED((ACC_ROWS, D), jnp.float32),
            pltpu.SemaphoreType.DMA,
            pltpu.SemaphoreType.DMA,
            pltpu.SemaphoreType.DMA,
        ],
    )
    return fn(x, eproj, srcs2d, dsts2d)


def _mlp_body(eps_ref, x_ref, a0_ref, a1_ref, w1_ref, b1_ref, w2_ref, b2_ref,
              lng_ref, lnb_ref, o_ref):
    h = x_ref[...] * eps_ref[0] + a0_ref[...] + a1_ref[...]
    h1 = jnp.dot(h, w1_ref[...], preferred_element_type=jnp.float32) + b1_ref[...]
    h1 = jnp.maximum(h1, 0.0)
    h2 = jnp.dot(h1, w2_ref[...], preferred_element_type=jnp.float32) + b2_ref[...]
    mu = jnp.mean(h2, axis=1, keepdims=True)
    dc = h2 - mu
    var = jnp.mean(dc * dc, axis=1, keepdims=True)
    hn = dc * lax.rsqrt(var + 1e-5) * lng_ref[...] + lnb_ref[...]
    o_ref[...] = jnp.maximum(hn + x_ref[...], 0.0)


def _node_mlp(epsv, x, a0, a1, W1f, b1f, W2, b2, ln_gamma, ln_beta):
    BN = 1000
    grid = (N // BN,)
    vec = lambda a: a.reshape(1, D)
    return pl.pallas_call(
        _mlp_body,
        grid=grid,
        in_specs=[
            pl.BlockSpec(memory_space=pltpu.SMEM),
            pl.BlockSpec((BN, D), lambda i: (i, 0)),
            pl.BlockSpec((BN, D), lambda i: (i, 0)),
            pl.BlockSpec((BN, D), lambda i: (i, 0)),
            pl.BlockSpec((D, D), lambda i: (0, 0)),
            pl.BlockSpec((1, D), lambda i: (0, 0)),
            pl.BlockSpec((D, D), lambda i: (0, 0)),
            pl.BlockSpec((1, D), lambda i: (0, 0)),
            pl.BlockSpec((1, D), lambda i: (0, 0)),
            pl.BlockSpec((1, D), lambda i: (0, 0)),
        ],
        out_specs=pl.BlockSpec((BN, D), lambda i: (i, 0)),
        out_shape=jax.ShapeDtypeStruct((N, D), jnp.float32),
    )(epsv, x, a0, a1, W1f, vec(b1f), W2, vec(b2), vec(ln_gamma), vec(ln_beta))


def kernel(x, edge_index, edge_attr, W_e, b_e, eps, W1, b1, bn_gamma, bn_beta,
           W2, b2, ln_gamma, ln_beta):
    src = edge_index[0].astype(jnp.int32)
    dst = edge_index[1].astype(jnp.int32)
    pad = E_PAD - E
    src_p = jnp.concatenate([src, jnp.zeros((pad,), jnp.int32)])
    dst_p = jnp.concatenate([dst, jnp.zeros((pad,), jnp.int32)])
    srcs2d = src_p.reshape(E_PAD // CHUNK, CHUNK)
    dsts2d = dst_p.reshape(E_PAD // CHUNK, CHUNK)
    ea_pad = jnp.concatenate(
        [edge_attr, jnp.zeros((pad, DE), edge_attr.dtype)], axis=0)

    eproj = _edge_proj(ea_pad, W_e, b_e)
    partials = _sc_aggregate(x, eproj, srcs2d, dsts2d)
    a0 = partials[0]
    a1 = partials[1]

    scale = bn_gamma / jnp.sqrt(1.0 + 1e-5)
    W1f = W1 * scale[None, :]
    b1f = b1 * scale + bn_beta
    epsv = (1.0 + eps).reshape(1).astype(jnp.float32)

    return _node_mlp(epsv, x, a0, a1, W1f, b1f, W2, b2, ln_gamma, ln_beta)


# speedup vs baseline: 1.8196x; 1.0017x over previous
import functools

import jax
import jax.numpy as jnp
from jax import lax
from jax.experimental import pallas as pl
from jax.experimental.pallas import tpu as pltpu, tpu_sc as plsc

N = 10000
D = 128
DE = 16
E = 320000

NC = 2
NS = 16
NW = NC * NS
CHUNK = 128
CPT = 80
E_PAD = CPT * CHUNK * NW
GRP = 8
NGRP = CPT // GRP
ACC_ROWS = N
ROWS_MAIN = 624
ROWS_LAST = N - 15 * ROWS_MAIN


def _eproj_body(ea_ref, we_ref, be_ref, o_ref):
    i = pl.program_id(0)
    e = (
        jnp.dot(ea_ref[...], we_ref[...], preferred_element_type=jnp.float32)
        + be_ref[...]
    )
    rows = lax.broadcasted_iota(jnp.int32, (e.shape[0], 1), 0) + i * e.shape[0]
    o_ref[...] = jnp.where(rows < E, e, -1e30)


def _edge_proj(ea_pad, W_e, b_e):
    BE = 1024
    grid = (E_PAD // BE,)
    return pl.pallas_call(
        _eproj_body,
        grid=grid,
        in_specs=[
            pl.BlockSpec((BE, DE), lambda i: (i, 0)),
            pl.BlockSpec((DE, D), lambda i: (0, 0)),
            pl.BlockSpec((1, D), lambda i: (0, 0)),
        ],
        out_specs=pl.BlockSpec((BE, D), lambda i: (i, 0)),
        out_shape=jax.ShapeDtypeStruct((E_PAD, D), jnp.float32),
    )(ea_pad, W_e, b_e.reshape(1, D))


def _sc_body(x_hbm, eproj_hbm, srcs_hbm, dsts_hbm, out_hbm,
             src8, dst8, e_buf, g_buf, agg_sh,
             sem_e, sem_g, sem_s):
    c = lax.axis_index("c")
    s = lax.axis_index("s")
    wid = s * NC + c

    z_buf = g_buf

    def _zrow(r, carry):
        for k in range(D // 16):
            z_buf[r, pl.ds(k * 16, 16)] = jnp.zeros((16,), jnp.float32)
        return carry

    lax.fori_loop(0, CHUNK, _zrow, 0)
    zbase = s * ROWS_MAIN

    @pl.when(s < NS - 1)
    def _():
        for k in range(4):
            cnt = CHUNK if k < 4 - 1 else ROWS_MAIN - 3 * CHUNK
            pltpu.sync_copy(z_buf.at[pl.ds(0, cnt)],
                            agg_sh.at[pl.ds(zbase + k * CHUNK, cnt)])

    @pl.when(s == NS - 1)
    def _():
        for k in range(5):
            cnt = CHUNK if k < 5 - 1 else ROWS_LAST - 4 * CHUNK
            pltpu.sync_copy(z_buf.at[pl.ds(0, cnt)],
                            agg_sh.at[pl.ds(zbase + k * CHUNK, cnt)])

    plsc.subcore_barrier()

    def _issue_g(src_row):
        pltpu.async_copy(x_hbm.at[src_row], g_buf, sem_g)

    def _wait_g(src_row):
        pltpu.make_async_copy(x_hbm.at[src_row], g_buf, sem_g).wait()

    def _eproj_rows(jj):
        return eproj_hbm.at[pl.ds((wid * CPT + jj) * CHUNK, CHUNK)]

    def _issue_e(jj):
        pltpu.async_copy(_eproj_rows(jj), e_buf, sem_e)

    def _wait_e(jj):
        pltpu.make_async_copy(_eproj_rows(jj), e_buf, sem_e).wait()

    def _wait_scat(dst_row):
        pltpu.make_async_copy(e_buf, agg_sh.at[dst_row], sem_s).wait()

    _issue_e(0)

    def _group(p, carry):
        @pl.when(p > 0)
        def _():
            _wait_scat(dst8.at[GRP - 1])
            _issue_e(p * GRP)

        off = wid * CPT + p * GRP
        pltpu.sync_copy(srcs_hbm.at[pl.ds(off, GRP)], src8)
        pltpu.sync_copy(dsts_hbm.at[pl.ds(off, GRP)], dst8)
        _issue_g(src8.at[0])

        for q in range(GRP):
            _wait_g(src8.at[q])
            _wait_e(p * GRP + q)

            def _row(r, rc):
                for rr in range(2):
                    for k in range(D // 16):
                        sl = pl.ds(k * 16, 16)
                        e_buf[2 * r + rr, sl] = jnp.maximum(
                            g_buf[2 * r + rr, sl] + e_buf[2 * r + rr, sl], 0.0)
                return rc

            lax.fori_loop(0, CHUNK // 2, _row, 0)

            pltpu.async_copy(e_buf, agg_sh.at[dst8.at[q]], sem_s, add=True)
            if q < GRP - 1:
                _issue_g(src8.at[q + 1])
                _wait_scat(dst8.at[q])
                _issue_e(p * GRP + q + 1)

        return carry

    lax.fori_loop(0, NGRP, _group, 0)
    _wait_scat(dst8.at[GRP - 1])
    plsc.subcore_barrier()

    @pl.when(s < NS - 1)
    def _():
        pltpu.sync_copy(agg_sh.at[pl.ds(zbase, ROWS_MAIN)],
                        out_hbm.at[c].at[pl.ds(zbase, ROWS_MAIN)])

    @pl.when(s == NS - 1)
    def _():
        pltpu.sync_copy(agg_sh.at[pl.ds(zbase, ROWS_LAST)],
                        out_hbm.at[c].at[pl.ds(zbase, ROWS_LAST)])


def _sc_aggregate(x, eproj, srcs2d, dsts2d):
    mesh = plsc.VectorSubcoreMesh(core_axis_name="c", subcore_axis_name="s",
                                  num_cores=NC, num_subcores=NS)
    fn = pl.kernel(
        _sc_body,
        out_type=jax.ShapeDtypeStruct((NC, N, D), jnp.float32),
        mesh=mesh,
        scratch_types=[
            pltpu.VMEM((GRP, CHUNK), jnp.int32),
            pltpu.VMEM((GRP, CHUNK), jnp.int32),
            pltpu.VMEM((CHUNK, D), jnp.float32),
            pltpu.VMEM((CHUNK, D), jnp.float32),
            pltpu.VMEM_SHARED((ACC_ROWS, D), jnp.float32),
            pltpu.SemaphoreType.DMA,
            pltpu.SemaphoreType.DMA,
            pltpu.SemaphoreType.DMA,
        ],
    )
    return fn(x, eproj, srcs2d, dsts2d)


def _mlp_body(eps_ref, x_ref, a0_ref, a1_ref, w1_ref, b1_ref, w2_ref, b2_ref,
              lng_ref, lnb_ref, o_ref):
    h = x_ref[...] * eps_ref[0] + a0_ref[...] + a1_ref[...]
    h1 = jnp.dot(h, w1_ref[...], preferred_element_type=jnp.float32) + b1_ref[...]
    h1 = jnp.maximum(h1, 0.0)
    h2 = jnp.dot(h1, w2_ref[...], preferred_element_type=jnp.float32) + b2_ref[...]
    mu = jnp.mean(h2, axis=1, keepdims=True)
    dc = h2 - mu
    var = jnp.mean(dc * dc, axis=1, keepdims=True)
    hn = dc * lax.rsqrt(var + 1e-5) * lng_ref[...] + lnb_ref[...]
    o_ref[...] = jnp.maximum(hn + x_ref[...], 0.0)


def _node_mlp(epsv, x, a0, a1, W1f, b1f, W2, b2, ln_gamma, ln_beta):
    BN = 1000
    grid = (N // BN,)
    vec = lambda a: a.reshape(1, D)
    return pl.pallas_call(
        _mlp_body,
        grid=grid,
        in_specs=[
            pl.BlockSpec(memory_space=pltpu.SMEM),
            pl.BlockSpec((BN, D), lambda i: (i, 0)),
            pl.BlockSpec((BN, D), lambda i: (i, 0)),
            pl.BlockSpec((BN, D), lambda i: (i, 0)),
            pl.BlockSpec((D, D), lambda i: (0, 0)),
            pl.BlockSpec((1, D), lambda i: (0, 0)),
            pl.BlockSpec((D, D), lambda i: (0, 0)),
            pl.BlockSpec((1, D), lambda i: (0, 0)),
            pl.BlockSpec((1, D), lambda i: (0, 0)),
            pl.BlockSpec((1, D), lambda i: (0, 0)),
        ],
        out_specs=pl.BlockSpec((BN, D), lambda i: (i, 0)),
        out_shape=jax.ShapeDtypeStruct((N, D), jnp.float32),
    )(epsv, x, a0, a1, W1f, vec(b1f), W2, vec(b2), vec(ln_gamma), vec(ln_beta))


def kernel(x, edge_index, edge_attr, W_e, b_e, eps, W1, b1, bn_gamma, bn_beta,
           W2, b2, ln_gamma, ln_beta):
    src = edge_index[0].astype(jnp.int32)
    dst = edge_index[1].astype(jnp.int32)
    pad = E_PAD - E
    src_p = jnp.concatenate([src, jnp.zeros((pad,), jnp.int32)])
    dst_p = jnp.concatenate([dst, jnp.zeros((pad,), jnp.int32)])
    srcs2d = src_p.reshape(E_PAD // CHUNK, CHUNK)
    dsts2d = dst_p.reshape(E_PAD // CHUNK, CHUNK)
    ea_pad = jnp.concatenate(
        [edge_attr, jnp.zeros((pad, DE), edge_attr.dtype)], axis=0)

    eproj = _edge_proj(ea_pad, W_e, b_e)
    partials = _sc_aggregate(x, eproj, srcs2d, dsts2d)
    a0 = partials[0]
    a1 = partials[1]

    scale = bn_gamma / jnp.sqrt(1.0 + 1e-5)
    W1f = W1 * scale[None, :]
    b1f = b1 * scale + bn_beta
    epsv = (1.0 + eps).reshape(1).astype(jnp.float32)

    return _node_mlp(epsv, x, a0, a1, W1f, b1f, W2, b2, ln_gamma, ln_beta)
